# baseline (speedup 1.0000x reference)
import functools

import jax
import jax.numpy as jnp
from jax import lax
from jax.experimental import pallas as pl
from jax.experimental.pallas import tpu as pltpu
from jax.experimental.pallas import tpu_sc as plsc

BATCH = 16384
EMBED = 64
NUM_CORES = 2
NUM_SUBCORES = 16
NW = NUM_CORES * NUM_SUBCORES
B_PER_W = BATCH // NW
CHUNK = 128
NCHUNK = B_PER_W // CHUNK


def _sbpr_body(idx_hbm, user_tab, item_tab, out_u, out_p, out_n,
               idx_v, rows_v, sem):
    wid = lax.axis_index("s") * NUM_CORES + lax.axis_index("c")
    base = wid * B_PER_W

    pltpu.sync_copy(idx_hbm.at[wid], idx_v)

    copies = []
    for k, tab in ((0, user_tab), (1, item_tab), (2, item_tab)):
        for c in range(NCHUNK):
            copies.append(
                pltpu.async_copy(
                    tab.at[idx_v.at[k, c]],
                    rows_v.at[k, pl.ds(c * CHUNK, CHUNK)],
                    sem,
                )
            )
    for cp in copies:
        cp.wait()

    pltpu.sync_copy(rows_v.at[0], out_u.at[pl.ds(base, B_PER_W)])
    pltpu.sync_copy(rows_v.at[1], out_p.at[pl.ds(base, B_PER_W)])
    pltpu.sync_copy(rows_v.at[2], out_n.at[pl.ds(base, B_PER_W)])


@jax.jit
def _sbpr(idx_all, embed_user, embed_item):
    out = jax.ShapeDtypeStruct((BATCH, EMBED), jnp.float32)
    mesh = plsc.VectorSubcoreMesh(core_axis_name="c", subcore_axis_name="s")
    return pl.kernel(
        _sbpr_body,
        out_type=(out, out, out),
        mesh=mesh,
        scratch_types=[
            pltpu.VMEM((3, NCHUNK, CHUNK), jnp.int32),
            pltpu.VMEM((3, B_PER_W, EMBED), jnp.float32),
            pltpu.SemaphoreType.DMA,
        ],
        compiler_params=pltpu.CompilerParams(
            use_tc_tiling_on_sc=False,
            has_side_effects=pltpu.SideEffectType.PURE,
        ),
    )(idx_all, embed_user, embed_item)


def kernel(batch_user, batch_pos_item, batch_neg_item, embed_user, embed_item):
    idx_all = (
        jnp.stack([batch_user, batch_pos_item, batch_neg_item])
        .reshape(3, NW, NCHUNK, CHUNK)
        .transpose(1, 0, 2, 3)
    )
    out_u, out_p, out_n = _sbpr(idx_all, embed_user, embed_item)
    return out_u, out_p, out_n


# speedup vs baseline: 1.6275x; 1.6275x over previous
import functools

import jax
import jax.numpy as jnp
from jax import lax
from jax.experimental import pallas as pl
from jax.experimental.pallas import tpu as pltpu
from jax.experimental.pallas import tpu_sc as plsc

BATCH = 16384
EMBED = 64
NUM_CORES = 2
NUM_SUBCORES = 16
NW = NUM_CORES * NUM_SUBCORES
B_PER_W = BATCH // NW
HALF = B_PER_W // 2


def _sbpr_body(idx_hbm, user_tab, item_tab,
               out_u, out_p, out_n, idx_v, rows_v, sem0, sem1, sem2):
    wid = lax.axis_index("s") * NUM_CORES + lax.axis_index("c")
    base = wid * B_PER_W

    pltpu.sync_copy(idx_hbm.at[wid], idx_v)

    tabs = (user_tab, item_tab, item_tab)
    sems = (sem0, sem1, sem2)
    outs = (out_u, out_p, out_n)

    for h in range(2):
        for k in range(3):
            def issue(g, carry, tab=tabs[k], sem=sems[k], k=k, h=h):
                vec = idx_v[pl.ds(k * B_PER_W + h * HALF + g * 16, 16)]
                for l in range(16):
                    row = vec[l]
                    pltpu.async_copy(
                        tab.at[row], rows_v.at[k, g * 16 + l], sem
                    )
                return carry

            lax.fori_loop(0, HALF // 16, issue, 0)

        for k in range(3):
            pltpu.make_async_copy(
                tabs[k].at[pl.ds(0, HALF)], rows_v.at[k], sems[k]
            ).wait()
            pltpu.sync_copy(
                rows_v.at[k],
                outs[k].at[pl.ds(base + h * HALF, HALF)],
            )


@jax.jit
def _sbpr(idx_all, embed_user, embed_item):
    out = jax.ShapeDtypeStruct((BATCH, EMBED), jnp.float32)
    mesh = plsc.VectorSubcoreMesh(core_axis_name="c", subcore_axis_name="s")
    return pl.kernel(
        _sbpr_body,
        out_type=(out, out, out),
        mesh=mesh,
        scratch_types=[
            pltpu.VMEM((3 * B_PER_W,), jnp.int32),
            pltpu.VMEM((3, HALF, EMBED), jnp.float32),
            pltpu.SemaphoreType.DMA,
            pltpu.SemaphoreType.DMA,
            pltpu.SemaphoreType.DMA,
        ],
    )(idx_all, embed_user, embed_item)


def kernel(batch_user, batch_pos_item, batch_neg_item, embed_user, embed_item):
    idx_all = (
        jnp.stack([batch_user, batch_pos_item, batch_neg_item])
        .reshape(3, NW, B_PER_W)
        .transpose(1, 0, 2)
        .reshape(NW, 3 * B_PER_W)
    )
    return _sbpr(idx_all, embed_user, embed_item)
